# baseline (speedup 1.0000x reference)
import functools

import jax
import jax.numpy as jnp
import numpy as np
from jax import lax
from jax.experimental import pallas as pl
from jax.experimental.pallas import tpu as pltpu
from jax.experimental.pallas import tpu_sc as plsc

N = 10000
NPAD = 10240
D_IN = 128
H1 = 64
H2 = 32
OUT = 2

NC = 2
NS = 16
NW = NC * NS
CHUNK = 128
U_DEG = 8
ROWS_PER_SUB = NPAD // NS

_SC_PARAMS = pltpu.CompilerParams(use_tc_tiling_on_sc=False)


def _worker_id():
    return lax.axis_index("s") * NC + lax.axis_index("c")


def _deg_body(chunks_per_worker, eidx_hbm, out_hbm,
              eidx_all, ones_v, zer_v, ssem, acc_sh):
    cid = lax.axis_index("c")
    sid = lax.axis_index("s")
    wid = _worker_id()
    for i in range(CHUNK // 16):
        ones_v[pl.ds(i * 16, 16)] = jnp.ones((16,), jnp.float32)
    for i in range(ROWS_PER_SUB // 16):
        zer_v[pl.ds(i * 16, 16)] = jnp.zeros((16,), jnp.float32)
    pltpu.sync_copy(
        eidx_hbm.at[pl.ds(wid * chunks_per_worker, chunks_per_worker)],
        eidx_all)
    pltpu.sync_copy(zer_v, acc_sh.at[pl.ds(sid * ROWS_PER_SUB, ROWS_PER_SUB)])
    plsc.subcore_barrier()

    def group(g, carry):
        scats = [
            pltpu.async_copy(
                ones_v, acc_sh.at[eidx_all.at[g * U_DEG + b, 1]], ssem.at[b],
                add=True)
            for b in range(U_DEG)
        ]
        for b in range(U_DEG):
            scats[b].wait()
        return carry

    lax.fori_loop(0, chunks_per_worker // U_DEG, group, 0)
    plsc.subcore_barrier()
    sl = pl.ds(sid * ROWS_PER_SUB, ROWS_PER_SUB)
    pltpu.sync_copy(acc_sh.at[sl], out_hbm.at[cid, sl])


def _make_deg_kernel(n_chunks):
    chunks_per_worker = n_chunks // NW
    mesh = plsc.VectorSubcoreMesh(core_axis_name="c", subcore_axis_name="s")
    return pl.kernel(
        functools.partial(_deg_body, chunks_per_worker),
        out_type=jax.ShapeDtypeStruct((NC, NPAD), jnp.float32),
        mesh=mesh,
        scratch_types=[
            pltpu.VMEM((chunks_per_worker, 2, CHUNK), jnp.int32),
            pltpu.VMEM((CHUNK,), jnp.float32),
            pltpu.VMEM((ROWS_PER_SUB,), jnp.float32),
            pltpu.SemaphoreType.DMA((U_DEG,)),
            pltpu.VMEM_SHARED((NPAD,), jnp.float32),
        ],
        compiler_params=_SC_PARAMS,
    )


def _agg_body(chunks_per_worker, u, hp_hbm, eidx_hbm, out_hbm,
              eidx_all, rows_v, gsem, ssem, acc_sh):
    cid = lax.axis_index("c")
    sid = lax.axis_index("s")
    wid = _worker_id()
    sl = pl.ds(sid * ROWS_PER_SUB, ROWS_PER_SUB)
    csl = pl.ds(wid * chunks_per_worker, chunks_per_worker)
    pltpu.sync_copy(eidx_hbm.at[csl], eidx_all)

    @pl.when(cid == 0)
    def _():
        pltpu.sync_copy(hp_hbm.at[sl], acc_sh.at[sl])

    @pl.when(cid == 1)
    def _():
        h = rows_v.shape[2]

        def zrow(i, carry):
            for j in range(h // 16):
                rows_v[0, i, pl.ds(j * 16, 16)] = jnp.zeros((16,),
                                                            jnp.float32)
            return carry

        lax.fori_loop(0, CHUNK, zrow, 0)
        for r in range(ROWS_PER_SUB // CHUNK):
            pltpu.sync_copy(
                rows_v.at[0],
                acc_sh.at[pl.ds(sid * ROWS_PER_SUB + r * CHUNK, CHUNK)])

    plsc.subcore_barrier()

    def group(g, carry):
        gats = [
            pltpu.async_copy(
                hp_hbm.at[eidx_all.at[g * u + b, 0]], rows_v.at[b], gsem.at[b])
            for b in range(u)
        ]
        scats = []
        for b in range(u):
            gats[b].wait()
            scats.append(pltpu.async_copy(
                rows_v.at[b], acc_sh.at[eidx_all.at[g * u + b, 1]],
                ssem.at[b], add=True))
        for b in range(u):
            scats[b].wait()
        return carry

    lax.fori_loop(0, chunks_per_worker // u, group, 0)
    plsc.subcore_barrier()
    pltpu.sync_copy(acc_sh.at[sl], out_hbm.at[cid, sl])


def _make_agg_kernel(n_chunks, h, u):
    chunks_per_worker = n_chunks // NW
    mesh = plsc.VectorSubcoreMesh(core_axis_name="c", subcore_axis_name="s")
    return pl.kernel(
        functools.partial(_agg_body, chunks_per_worker, u),
        out_type=jax.ShapeDtypeStruct((NC, NPAD, h), jnp.float32),
        mesh=mesh,
        scratch_types=[
            pltpu.VMEM((chunks_per_worker, 2, CHUNK), jnp.int32),
            pltpu.VMEM((u, CHUNK, h), jnp.float32),
            pltpu.SemaphoreType.DMA((u,)),
            pltpu.SemaphoreType.DMA((u,)),
            pltpu.VMEM_SHARED((NPAD, h), jnp.float32),
        ],
        compiler_params=_SC_PARAMS,
    )


def _tc1a_body(x_ref, w1_ref, h1_ref):
    xp = jnp.concatenate(
        [x_ref[...], jnp.zeros((NPAD - N, D_IN), jnp.float32)], axis=0)
    h1_ref[...] = jnp.dot(xp, w1_ref[...], preferred_element_type=jnp.float32)


def _tc1b_body(h1_ref, degp_ref, hp_ref, dinv_ref):
    deg = degp_ref[:, 0:1] + degp_ref[:, 1:2] + 1.0
    rid = lax.broadcasted_iota(jnp.int32, (NPAD, 1), 0)
    dinv = jnp.where(rid < N, lax.rsqrt(deg), 0.0)
    dinv_ref[...] = dinv
    hp_ref[...] = h1_ref[...] * dinv


def _bn_relu(t, g, b):
    m = jnp.sum(t, axis=0, keepdims=True) / N
    v = jnp.sum(t * t, axis=0, keepdims=True) / N - m * m
    return jnp.maximum((t - m) * lax.rsqrt(v + 1e-5) * g + b, 0.0)


def _tc2_body(p_ref, dinv_ref, g1_ref, be1_ref, w2_ref, hp2_ref):
    t = p_ref[0] + p_ref[1]
    agg = t * dinv_ref[...]
    a1 = _bn_relu(agg, g1_ref[...], be1_ref[...])
    hp2_ref[...] = jnp.dot(a1, w2_ref[...],
                           preferred_element_type=jnp.float32) * dinv_ref[...]


def _tc3_body(p_ref, dinv_ref, g2_ref, be2_ref, wfc_ref, bfc_ref,
              out_ref):
    t = p_ref[0] + p_ref[1]
    agg = t * dinv_ref[...]
    a2 = _bn_relu(agg, g2_ref[...], be2_ref[...])
    out_ref[...] = jnp.dot(a2, wfc_ref[...],
                           preferred_element_type=jnp.float32) + bfc_ref[...]


def kernel(x, edge_index, W1, b1, g1, be1, W2, b2, g2, be2, Wfc, bfc):
    e = edge_index.shape[1]
    egrp = NW * CHUNK * 8
    epad = ((e + egrp - 1) // egrp) * egrp
    n_chunks = epad // CHUNK
    if e % CHUNK == 0:
        eidx0 = edge_index.reshape(2, e // CHUNK, CHUNK).transpose(1, 0, 2)
        padc = n_chunks - e // CHUNK
        if padc:
            cpad = np.broadcast_to(
                (N + np.arange(padc * CHUNK) % (NPAD - N)).astype(np.int32)
                .reshape(padc, 1, CHUNK), (padc, 2, CHUNK))
            eidx = jnp.concatenate([eidx0, jnp.asarray(cpad)], axis=0)
        else:
            eidx = eidx0
    else:
        pad = epad - e
        fill = jnp.asarray(N + np.arange(pad) % (NPAD - N), dtype=jnp.int32)
        src = jnp.concatenate([edge_index[0], fill])
        dst = jnp.concatenate([edge_index[1], fill])
        eidx = jnp.stack([src.reshape(n_chunks, CHUNK),
                          dst.reshape(n_chunks, CHUNK)], axis=1)

    degp = _make_deg_kernel(n_chunks)(eidx)

    h1 = pl.pallas_call(
        _tc1a_body,
        out_shape=jax.ShapeDtypeStruct((NPAD, H1), jnp.float32),
    )(x, W1)

    hp1, dinv = pl.pallas_call(
        _tc1b_body,
        out_shape=[
            jax.ShapeDtypeStruct((NPAD, H1), jnp.float32),
            jax.ShapeDtypeStruct((NPAD, 1), jnp.float32),
        ],
    )(h1, degp.T)

    p1 = _make_agg_kernel(n_chunks, H1, 8)(hp1, eidx)

    hp2 = pl.pallas_call(
        _tc2_body,
        out_shape=jax.ShapeDtypeStruct((NPAD, H2), jnp.float32),
    )(p1, dinv, g1, be1, W2)

    p2 = _make_agg_kernel(n_chunks, H2, 8)(hp2, eidx)

    logits = pl.pallas_call(
        _tc3_body,
        out_shape=jax.ShapeDtypeStruct((NPAD, OUT), jnp.float32),
    )(p2, dinv, g2, be2, Wfc, bfc)

    return logits[:N]


# speedup vs baseline: 1.0065x; 1.0065x over previous
import functools

import jax
import jax.numpy as jnp
import numpy as np
from jax import lax
from jax.experimental import pallas as pl
from jax.experimental.pallas import tpu as pltpu
from jax.experimental.pallas import tpu_sc as plsc

N = 10000
NPAD = 10240
D_IN = 128
H1 = 64
H2 = 32
OUT = 2

NC = 2
NS = 16
NW = NC * NS
CHUNK = 128
U_DEG = 8
ROWS_PER_SUB = NPAD // NS

_SC_PARAMS = pltpu.CompilerParams(use_tc_tiling_on_sc=False)


def _worker_id():
    return lax.axis_index("s") * NC + lax.axis_index("c")


def _deg_body(chunks_per_worker, eidx_hbm, out_hbm,
              eidx_all, ones_v, zer_v, ssem, acc_sh):
    cid = lax.axis_index("c")
    sid = lax.axis_index("s")
    wid = _worker_id()
    for i in range(CHUNK // 16):
        ones_v[pl.ds(i * 16, 16)] = jnp.ones((16,), jnp.float32)
    for i in range(ROWS_PER_SUB // 16):
        zer_v[pl.ds(i * 16, 16)] = jnp.zeros((16,), jnp.float32)
    pltpu.sync_copy(
        eidx_hbm.at[pl.ds(wid * chunks_per_worker, chunks_per_worker)],
        eidx_all)
    pltpu.sync_copy(zer_v, acc_sh.at[pl.ds(sid * ROWS_PER_SUB, ROWS_PER_SUB)])
    plsc.subcore_barrier()

    def group(g, carry):
        scats = [
            pltpu.async_copy(
                ones_v, acc_sh.at[eidx_all.at[g * U_DEG + b, 1]], ssem.at[b],
                add=True)
            for b in range(U_DEG)
        ]
        for b in range(U_DEG):
            scats[b].wait()
        return carry

    lax.fori_loop(0, chunks_per_worker // U_DEG, group, 0)
    plsc.subcore_barrier()
    sl = pl.ds(sid * ROWS_PER_SUB, ROWS_PER_SUB)
    pltpu.sync_copy(acc_sh.at[sl], out_hbm.at[cid, sl])


def _make_deg_kernel(n_chunks):
    chunks_per_worker = n_chunks // NW
    mesh = plsc.VectorSubcoreMesh(core_axis_name="c", subcore_axis_name="s")
    return pl.kernel(
        functools.partial(_deg_body, chunks_per_worker),
        out_type=jax.ShapeDtypeStruct((NC, NPAD), jnp.float32),
        mesh=mesh,
        scratch_types=[
            pltpu.VMEM((chunks_per_worker, 2, CHUNK), jnp.int32),
            pltpu.VMEM((CHUNK,), jnp.float32),
            pltpu.VMEM((ROWS_PER_SUB,), jnp.float32),
            pltpu.SemaphoreType.DMA((U_DEG,)),
            pltpu.VMEM_SHARED((NPAD,), jnp.float32),
        ],
        compiler_params=_SC_PARAMS,
    )


def _agg_body(chunks_per_worker, u, hp_hbm, eidx_hbm, out_hbm,
              eidx_all, rows_v, gsem, ssem, acc_sh):
    cid = lax.axis_index("c")
    sid = lax.axis_index("s")
    wid = _worker_id()
    sl = pl.ds(sid * ROWS_PER_SUB, ROWS_PER_SUB)
    csl = pl.ds(wid * chunks_per_worker, chunks_per_worker)
    pltpu.sync_copy(eidx_hbm.at[csl], eidx_all)

    @pl.when(cid == 0)
    def _():
        pltpu.sync_copy(hp_hbm.at[sl], acc_sh.at[sl])

    @pl.when(cid == 1)
    def _():
        h = rows_v.shape[2]

        def zrow(i, carry):
            for j in range(h // 16):
                rows_v[0, i, pl.ds(j * 16, 16)] = jnp.zeros((16,),
                                                            jnp.float32)
            return carry

        lax.fori_loop(0, CHUNK, zrow, 0)
        for r in range(ROWS_PER_SUB // CHUNK):
            pltpu.sync_copy(
                rows_v.at[0],
                acc_sh.at[pl.ds(sid * ROWS_PER_SUB + r * CHUNK, CHUNK)])

    plsc.subcore_barrier()

    def group(g, carry):
        gats = [
            pltpu.async_copy(
                hp_hbm.at[eidx_all.at[g * u + b, 0]], rows_v.at[b], gsem.at[b])
            for b in range(u)
        ]
        scats = []
        for b in range(u):
            gats[b].wait()
            scats.append(pltpu.async_copy(
                rows_v.at[b], acc_sh.at[eidx_all.at[g * u + b, 1]],
                ssem.at[b], add=True))
        for b in range(u):
            scats[b].wait()
        return carry

    lax.fori_loop(0, chunks_per_worker // u, group, 0)
    plsc.subcore_barrier()
    pltpu.sync_copy(acc_sh.at[sl], out_hbm.at[cid, sl])


def _make_agg_kernel(n_chunks, h, u):
    chunks_per_worker = n_chunks // NW
    mesh = plsc.VectorSubcoreMesh(core_axis_name="c", subcore_axis_name="s")
    return pl.kernel(
        functools.partial(_agg_body, chunks_per_worker, u),
        out_type=jax.ShapeDtypeStruct((NC, NPAD, h), jnp.float32),
        mesh=mesh,
        scratch_types=[
            pltpu.VMEM((chunks_per_worker, 2, CHUNK), jnp.int32),
            pltpu.VMEM((u, CHUNK, h), jnp.float32),
            pltpu.SemaphoreType.DMA((u,)),
            pltpu.SemaphoreType.DMA((u,)),
            pltpu.VMEM_SHARED((NPAD, h), jnp.float32),
        ],
        compiler_params=_SC_PARAMS,
    )


def _tc1_body(x_ref, w1_ref, degp_ref, hp_ref, dinv_ref):
    deg = degp_ref[:, 0:1] + degp_ref[:, 1:2] + 1.0
    rid = lax.broadcasted_iota(jnp.int32, (NPAD, 1), 0)
    dinv = jnp.where(rid < N, lax.rsqrt(deg), 0.0)
    dinv_ref[...] = dinv
    xp = jnp.concatenate(
        [x_ref[...], jnp.zeros((NPAD - N, D_IN), jnp.float32)], axis=0)
    hp_ref[...] = jnp.dot(xp, w1_ref[...],
                          preferred_element_type=jnp.float32) * dinv


def _bn_relu(t, g, b):
    m = jnp.sum(t, axis=0, keepdims=True) / N
    v = jnp.sum(t * t, axis=0, keepdims=True) / N - m * m
    return jnp.maximum((t - m) * lax.rsqrt(v + 1e-5) * g + b, 0.0)


def _tc2_body(p_ref, dinv_ref, g1_ref, be1_ref, w2_ref, hp2_ref):
    t = p_ref[0] + p_ref[1]
    agg = t * dinv_ref[...]
    a1 = _bn_relu(agg, g1_ref[...], be1_ref[...])
    hp2_ref[...] = jnp.dot(a1, w2_ref[...],
                           preferred_element_type=jnp.float32) * dinv_ref[...]


def _tc3_body(p_ref, dinv_ref, g2_ref, be2_ref, wfc_ref, bfc_ref,
              out_ref):
    t = p_ref[0] + p_ref[1]
    agg = t * dinv_ref[...]
    a2 = _bn_relu(agg, g2_ref[...], be2_ref[...])
    out_ref[...] = jnp.dot(a2, wfc_ref[...],
                           preferred_element_type=jnp.float32) + bfc_ref[...]


def kernel(x, edge_index, W1, b1, g1, be1, W2, b2, g2, be2, Wfc, bfc):
    e = edge_index.shape[1]
    egrp = NW * CHUNK * 8
    epad = ((e + egrp - 1) // egrp) * egrp
    n_chunks = epad // CHUNK
    if e % CHUNK == 0:
        eidx0 = edge_index.reshape(2, e // CHUNK, CHUNK).transpose(1, 0, 2)
        padc = n_chunks - e // CHUNK
        if padc:
            cpad = np.broadcast_to(
                (N + np.arange(padc * CHUNK) % (NPAD - N)).astype(np.int32)
                .reshape(padc, 1, CHUNK), (padc, 2, CHUNK))
            eidx = jnp.concatenate([eidx0, jnp.asarray(cpad)], axis=0)
        else:
            eidx = eidx0
    else:
        pad = epad - e
        fill = jnp.asarray(N + np.arange(pad) % (NPAD - N), dtype=jnp.int32)
        src = jnp.concatenate([edge_index[0], fill])
        dst = jnp.concatenate([edge_index[1], fill])
        eidx = jnp.stack([src.reshape(n_chunks, CHUNK),
                          dst.reshape(n_chunks, CHUNK)], axis=1)

    degp = _make_deg_kernel(n_chunks)(eidx)

    hp1, dinv = pl.pallas_call(
        _tc1_body,
        out_shape=[
            jax.ShapeDtypeStruct((NPAD, H1), jnp.float32),
            jax.ShapeDtypeStruct((NPAD, 1), jnp.float32),
        ],
    )(x, W1, degp.T)

    p1 = _make_agg_kernel(n_chunks, H1, 8)(hp1, eidx)

    hp2 = pl.pallas_call(
        _tc2_body,
        out_shape=jax.ShapeDtypeStruct((NPAD, H2), jnp.float32),
    )(p1, dinv, g1, be1, W2)

    p2 = _make_agg_kernel(n_chunks, H2, 8)(hp2, eidx)

    logits = pl.pallas_call(
        _tc3_body,
        out_shape=jax.ShapeDtypeStruct((NPAD, OUT), jnp.float32),
    )(p2, dinv, g2, be2, Wfc, bfc)

    return logits[:N]
